# baseline (device time: 35763 ns/iter reference)
import jax
import jax.numpy as jnp
from jax import lax
from jax.experimental import pallas as pl
from jax.experimental.pallas import tpu as pltpu

N_DEV = 4
N_LAYERS = 3


def kernel(x, Win0, Wout0, Win1, Wout1, Win2, Wout2):
    b, d = x.shape

    def body(x_ref, win0_ref, wout0_ref, win1_ref, wout1_ref, win2_ref,
             wout2_ref, out_ref, comm_ref, p_ref, send_sems, recv_sems):
        my_pos = lax.axis_index("i")

        barrier_sem = pltpu.get_barrier_semaphore()
        for k in range(1, N_DEV):
            pl.semaphore_signal(
                barrier_sem, inc=1,
                device_id=((my_pos + k) % N_DEV,),
                device_id_type=pl.DeviceIdType.MESH,
            )
        pl.semaphore_wait(barrier_sem, N_DEV - 1)

        wins = [win0_ref, win1_ref, win2_ref]
        wouts = [wout0_ref, wout1_ref, wout2_ref]

        xv = x_ref[:, :]
        for l in range(N_LAYERS):
            h = jnp.maximum(
                jnp.dot(xv, wins[l][:, :], preferred_element_type=jnp.float32),
                0.0,
            )
            p = jnp.dot(h, wouts[l][:, :], preferred_element_type=jnp.float32)
            p_ref[l, :, :] = p

            sends = []
            for k in range(1, N_DEV):
                rdma = pltpu.make_async_remote_copy(
                    src_ref=p_ref.at[l],
                    dst_ref=comm_ref.at[l, N_DEV - 1 - k],
                    send_sem=send_sems.at[l, k - 1],
                    recv_sem=recv_sems.at[l, N_DEV - 1 - k],
                    device_id=((my_pos + k) % N_DEV,),
                    device_id_type=pl.DeviceIdType.MESH,
                )
                rdma.start()
                sends.append(rdma)

            xv = p
            for m in range(N_DEV - 1):
                recv = pltpu.make_async_remote_copy(
                    src_ref=p_ref.at[l],
                    dst_ref=comm_ref.at[l, m],
                    send_sem=send_sems.at[l, 0],
                    recv_sem=recv_sems.at[l, m],
                    device_id=(my_pos,),
                    device_id_type=pl.DeviceIdType.MESH,
                )
                recv.wait_recv()
                xv = xv + comm_ref[l, m]

            for rdma in sends:
                rdma.wait_send()

        out_ref[:, :] = xv

    return pl.pallas_call(
        body,
        out_shape=jax.ShapeDtypeStruct((b, d), jnp.float32),
        in_specs=[pl.BlockSpec(memory_space=pltpu.VMEM)] * 7,
        out_specs=pl.BlockSpec(memory_space=pltpu.VMEM),
        scratch_shapes=[
            pltpu.VMEM((N_LAYERS, N_DEV - 1, b, d), jnp.float32),
            pltpu.VMEM((N_LAYERS, b, d), jnp.float32),
            pltpu.SemaphoreType.DMA((N_LAYERS, N_DEV - 1)),
            pltpu.SemaphoreType.DMA((N_LAYERS, N_DEV - 1)),
        ],
        compiler_params=pltpu.CompilerParams(collective_id=0),
    )(x, Win0, Wout0, Win1, Wout1, Win2, Wout2)


# device time: 32126 ns/iter; 1.1132x vs baseline; 1.1132x over previous
import jax
import jax.numpy as jnp
from jax import lax
from jax.experimental import pallas as pl
from jax.experimental.pallas import tpu as pltpu

N_DEV = 4
N_LAYERS = 3
R = 2


def kernel(x, Win0, Wout0, Win1, Wout1, Win2, Wout2):
    b, d = x.shape
    ch = b // R

    def body(x_ref, win0_ref, wout0_ref, win1_ref, wout1_ref, win2_ref,
             wout2_ref, out_ref, comm_a, comm_b, p_ref, sa_ref,
             send_a, recv_a, send_b, recv_b):
        my_pos = lax.axis_index("i")
        partner_a = my_pos ^ 1
        partner_b = 3 - my_pos

        barrier_sem = pltpu.get_barrier_semaphore()
        for pid in (partner_a, partner_b):
            pl.semaphore_signal(
                barrier_sem, inc=1,
                device_id=(pid,), device_id_type=pl.DeviceIdType.MESH,
            )
        pl.semaphore_wait(barrier_sem, 2)

        wins = [win0_ref, win1_ref, win2_ref]
        wouts = [wout0_ref, wout1_ref, wout2_ref]
        rdma_a = {}
        rdma_b = {}

        def start_a(l, r):
            dsc = pltpu.make_async_remote_copy(
                src_ref=p_ref.at[l, r],
                dst_ref=comm_a.at[l, r],
                send_sem=send_a.at[l, r],
                recv_sem=recv_a.at[l, r],
                device_id=(partner_a,),
                device_id_type=pl.DeviceIdType.MESH,
            )
            dsc.start()
            rdma_a[(l, r)] = dsc

        def finish_a_start_b(l, r):
            rdma_a[(l, r)].wait_recv()
            sa_ref[l, r] = p_ref[l, r] + comm_a[l, r]
            dsc = pltpu.make_async_remote_copy(
                src_ref=sa_ref.at[l, r],
                dst_ref=comm_b.at[l, r],
                send_sem=send_b.at[l, r],
                recv_sem=recv_b.at[l, r],
                device_id=(partner_b,),
                device_id_type=pl.DeviceIdType.MESH,
            )
            dsc.start()
            rdma_b[(l, r)] = dsc

        for l in range(N_LAYERS):
            for r in range(R):
                if l == 0:
                    xc = x_ref[pl.ds(r * ch, ch), :]
                else:
                    rdma_b[(l - 1, r)].wait_recv()
                    xc = sa_ref[l - 1, r] + comm_b[l - 1, r]
                h = jnp.maximum(
                    jnp.dot(xc, wins[l][:, :],
                            preferred_element_type=jnp.float32),
                    0.0,
                )
                p_ref[l, r] = jnp.dot(
                    h, wouts[l][:, :], preferred_element_type=jnp.float32
                )
                start_a(l, r)
                if r > 0:
                    finish_a_start_b(l, r - 1)
            finish_a_start_b(l, R - 1)

        for r in range(R):
            rdma_b[(N_LAYERS - 1, r)].wait_recv()
            out_ref[pl.ds(r * ch, ch), :] = (
                sa_ref[N_LAYERS - 1, r] + comm_b[N_LAYERS - 1, r]
            )

        for dsc in list(rdma_a.values()) + list(rdma_b.values()):
            dsc.wait_send()

    return pl.pallas_call(
        body,
        out_shape=jax.ShapeDtypeStruct((b, d), jnp.float32),
        in_specs=[pl.BlockSpec(memory_space=pltpu.VMEM)] * 7,
        out_specs=pl.BlockSpec(memory_space=pltpu.VMEM),
        scratch_shapes=[
            pltpu.VMEM((N_LAYERS, R, ch, d), jnp.float32),
            pltpu.VMEM((N_LAYERS, R, ch, d), jnp.float32),
            pltpu.VMEM((N_LAYERS, R, ch, d), jnp.float32),
            pltpu.VMEM((N_LAYERS, R, ch, d), jnp.float32),
            pltpu.SemaphoreType.DMA((N_LAYERS, R)),
            pltpu.SemaphoreType.DMA((N_LAYERS, R)),
            pltpu.SemaphoreType.DMA((N_LAYERS, R)),
            pltpu.SemaphoreType.DMA((N_LAYERS, R)),
        ],
        compiler_params=pltpu.CompilerParams(collective_id=0),
    )(x, Win0, Wout0, Win1, Wout1, Win2, Wout2)
